# baseline (speedup 1.0000x reference)
import jax
import jax.numpy as jnp
from jax.experimental import pallas as pl
from jax.experimental.pallas import tpu as pltpu

interpret = False

_G = 16


def _body(tw_ref, th_ref, tm_ref, node_ref, week_ref, hour_ref, minute_ref,
          xs_ref, xt_ref, xs_rep, xt_build, sem_xs, sem_xt):
    i = pl.program_id(0)
    nsteps = pl.num_programs(0)
    n, h = node_ref.shape
    slot = jax.lax.rem(i, 2)

    @pl.when(i == 0)
    def _():
        for r in range(_G):
            xs_rep[r * n:(r + 1) * n, :] = node_ref[...]

    @pl.when(i >= 2)
    def _():
        pltpu.make_async_copy(xs_rep, xs_ref.at[i - 2], sem_xs.at[slot]).wait()
    pltpu.make_async_copy(xs_rep, xs_ref.at[i], sem_xs.at[slot]).start()

    @pl.when(i >= 2)
    def _():
        pltpu.make_async_copy(xt_build.at[slot], xt_ref.at[i - 2],
                              sem_xt.at[slot]).wait()
    for g in range(_G):
        idx = i * _G + g
        row = (week_ref[pl.ds(tw_ref[idx], 1), :]
               + hour_ref[pl.ds(th_ref[idx], 1), :]
               + minute_ref[pl.ds(tm_ref[idx], 1), :])
        xt_build[slot, g * n:(g + 1) * n, :] = jnp.broadcast_to(row, (n, h))
    pltpu.make_async_copy(xt_build.at[slot], xt_ref.at[i],
                          sem_xt.at[slot]).start()

    @pl.when(i == nsteps - 1)
    def _():
        pltpu.make_async_copy(xt_build.at[1 - slot], xt_ref.at[i - 1],
                              sem_xt.at[1 - slot]).wait()
        pltpu.make_async_copy(xt_build.at[slot], xt_ref.at[i],
                              sem_xt.at[slot]).wait()
        pltpu.make_async_copy(xs_rep, xs_ref.at[i - 1],
                              sem_xs.at[1 - slot]).wait()
        pltpu.make_async_copy(xs_rep, xs_ref.at[i],
                              sem_xs.at[slot]).wait()


def kernel(t, node_embedding, week_table, hour_table, minute_table):
    B, L = t.shape[0], t.shape[1]
    N, H = node_embedding.shape
    wn, wd = week_table.shape
    hn, hd = hour_table.shape
    mn, md = minute_table.shape
    steps = (B * L) // _G

    week_p = jnp.zeros((8, H), jnp.float32).at[:wn, :wd].set(week_table)
    hour_p = jnp.zeros((24, H), jnp.float32).at[:hn, wd:wd + hd].set(hour_table)
    minute_p = jnp.zeros((8, H), jnp.float32).at[:mn, wd + hd:].set(minute_table)

    tw = t[:, :, 0, 1].reshape(-1).astype(jnp.int32)
    th = t[:, :, 0, 2].reshape(-1).astype(jnp.int32)
    tm = t[:, :, 0, 3].reshape(-1).astype(jnp.int32)

    grid_spec = pltpu.PrefetchScalarGridSpec(
        num_scalar_prefetch=3,
        grid=(steps,),
        in_specs=[
            pl.BlockSpec((N, H), lambda i, *_: (0, 0)),
            pl.BlockSpec((8, H), lambda i, *_: (0, 0)),
            pl.BlockSpec((24, H), lambda i, *_: (0, 0)),
            pl.BlockSpec((8, H), lambda i, *_: (0, 0)),
        ],
        out_specs=[
            pl.BlockSpec(memory_space=pl.ANY),
            pl.BlockSpec(memory_space=pl.ANY),
        ],
        scratch_shapes=[
            pltpu.VMEM((_G * N, H), jnp.float32),
            pltpu.VMEM((2, _G * N, H), jnp.float32),
            pltpu.SemaphoreType.DMA((2,)),
            pltpu.SemaphoreType.DMA((2,)),
        ],
    )
    xs, xt = pl.pallas_call(
        _body,
        grid_spec=grid_spec,
        out_shape=[jax.ShapeDtypeStruct((steps, _G * N, H), jnp.float32)] * 2,
        interpret=interpret,
    )(tw, th, tm, node_embedding, week_p, hour_p, minute_p)
    return xs.reshape(B, L, N, H), xt.reshape(B, L, N, H)


# speedup vs baseline: 1.0229x; 1.0229x over previous
import jax
import jax.numpy as jnp
from jax.experimental import pallas as pl
from jax.experimental.pallas import tpu as pltpu

interpret = False

_G = 8


def _body(tw_ref, th_ref, tm_ref, node_ref, week_ref, hour_ref, minute_ref,
          xs_ref, xt_ref, xs_rep, xt_build, sem_xs, sem_xt):
    i = pl.program_id(0)
    nsteps = pl.num_programs(0)
    n, h = node_ref.shape
    slot = jax.lax.rem(i, 2)

    @pl.when(i == 0)
    def _():
        for r in range(_G):
            xs_rep[r * n:(r + 1) * n, :] = node_ref[...]

    @pl.when(i >= 2)
    def _():
        pltpu.make_async_copy(xs_rep, xs_ref.at[i - 2], sem_xs.at[slot]).wait()
    pltpu.make_async_copy(xs_rep, xs_ref.at[i], sem_xs.at[slot]).start()

    @pl.when(i >= 2)
    def _():
        pltpu.make_async_copy(xt_build.at[slot], xt_ref.at[i - 2],
                              sem_xt.at[slot]).wait()
    for g in range(_G):
        idx = i * _G + g
        row = (week_ref[pl.ds(tw_ref[idx], 1), :]
               + hour_ref[pl.ds(th_ref[idx], 1), :]
               + minute_ref[pl.ds(tm_ref[idx], 1), :])
        xt_build[slot, g * n:(g + 1) * n, :] = jnp.broadcast_to(row, (n, h))
    pltpu.make_async_copy(xt_build.at[slot], xt_ref.at[i],
                          sem_xt.at[slot]).start()

    @pl.when(i == nsteps - 1)
    def _():
        pltpu.make_async_copy(xt_build.at[1 - slot], xt_ref.at[i - 1],
                              sem_xt.at[1 - slot]).wait()
        pltpu.make_async_copy(xt_build.at[slot], xt_ref.at[i],
                              sem_xt.at[slot]).wait()
        pltpu.make_async_copy(xs_rep, xs_ref.at[i - 1],
                              sem_xs.at[1 - slot]).wait()
        pltpu.make_async_copy(xs_rep, xs_ref.at[i],
                              sem_xs.at[slot]).wait()


def kernel(t, node_embedding, week_table, hour_table, minute_table):
    B, L = t.shape[0], t.shape[1]
    N, H = node_embedding.shape
    wn, wd = week_table.shape
    hn, hd = hour_table.shape
    mn, md = minute_table.shape
    steps = (B * L) // _G

    week_p = jnp.zeros((8, H), jnp.float32).at[:wn, :wd].set(week_table)
    hour_p = jnp.zeros((24, H), jnp.float32).at[:hn, wd:wd + hd].set(hour_table)
    minute_p = jnp.zeros((8, H), jnp.float32).at[:mn, wd + hd:].set(minute_table)

    tw = t[:, :, 0, 1].reshape(-1).astype(jnp.int32)
    th = t[:, :, 0, 2].reshape(-1).astype(jnp.int32)
    tm = t[:, :, 0, 3].reshape(-1).astype(jnp.int32)

    grid_spec = pltpu.PrefetchScalarGridSpec(
        num_scalar_prefetch=3,
        grid=(steps,),
        in_specs=[
            pl.BlockSpec((N, H), lambda i, *_: (0, 0)),
            pl.BlockSpec((8, H), lambda i, *_: (0, 0)),
            pl.BlockSpec((24, H), lambda i, *_: (0, 0)),
            pl.BlockSpec((8, H), lambda i, *_: (0, 0)),
        ],
        out_specs=[
            pl.BlockSpec(memory_space=pl.ANY),
            pl.BlockSpec(memory_space=pl.ANY),
        ],
        scratch_shapes=[
            pltpu.VMEM((_G * N, H), jnp.float32),
            pltpu.VMEM((2, _G * N, H), jnp.float32),
            pltpu.SemaphoreType.DMA((2,)),
            pltpu.SemaphoreType.DMA((2,)),
        ],
    )
    xs, xt = pl.pallas_call(
        _body,
        grid_spec=grid_spec,
        out_shape=[jax.ShapeDtypeStruct((steps, _G * N, H), jnp.float32)] * 2,
        interpret=interpret,
    )(tw, th, tm, node_embedding, week_p, hour_p, minute_p)
    return xs.reshape(B, L, N, H), xt.reshape(B, L, N, H)


# speedup vs baseline: 1.0272x; 1.0042x over previous
import jax
import jax.numpy as jnp
from jax.experimental import pallas as pl
from jax.experimental.pallas import tpu as pltpu

interpret = False

_G = 8
_Q = 2


def _body(tw_ref, th_ref, tm_ref, node_ref, week_ref, hour_ref, minute_ref,
          xs_ref, xt_ref, xs_rep, xt_build, sem_xs, sem_xt):
    i = pl.program_id(0)
    nsteps = pl.num_programs(0)
    n, h = node_ref.shape
    slot = jax.lax.rem(i, 2)
    rows = _G * n
    sub = rows // _Q

    @pl.when(i == 0)
    def _():
        for r in range(_G):
            xs_rep[r * n:(r + 1) * n, :] = node_ref[...]

    def xs_copy(step, q, s):
        return pltpu.make_async_copy(
            xs_rep.at[q * sub:(q + 1) * sub, :],
            xs_ref.at[step, q * sub:(q + 1) * sub, :],
            sem_xs.at[q, s])

    def xt_copy(step, q, s):
        return pltpu.make_async_copy(
            xt_build.at[s, q * sub:(q + 1) * sub, :],
            xt_ref.at[step, q * sub:(q + 1) * sub, :],
            sem_xt.at[q, s])

    @pl.when(i >= 2)
    def _():
        for q in range(_Q):
            xs_copy(i - 2, q, slot).wait()
    for q in range(_Q):
        xs_copy(i, q, slot).start()

    @pl.when(i >= 2)
    def _():
        for q in range(_Q):
            xt_copy(i - 2, q, slot).wait()
    for g in range(_G):
        idx = i * _G + g
        row = (week_ref[pl.ds(tw_ref[idx], 1), :]
               + hour_ref[pl.ds(th_ref[idx], 1), :]
               + minute_ref[pl.ds(tm_ref[idx], 1), :])
        xt_build[slot, g * n:(g + 1) * n, :] = jnp.broadcast_to(row, (n, h))
    for q in range(_Q):
        xt_copy(i, q, slot).start()

    @pl.when(i == nsteps - 1)
    def _():
        for q in range(_Q):
            xt_copy(i - 1, q, 1 - slot).wait()
            xt_copy(i, q, slot).wait()
            xs_copy(i - 1, q, 1 - slot).wait()
            xs_copy(i, q, slot).wait()


def kernel(t, node_embedding, week_table, hour_table, minute_table):
    B, L = t.shape[0], t.shape[1]
    N, H = node_embedding.shape
    wn, wd = week_table.shape
    hn, hd = hour_table.shape
    mn, md = minute_table.shape
    steps = (B * L) // _G

    week_p = jnp.zeros((8, H), jnp.float32).at[:wn, :wd].set(week_table)
    hour_p = jnp.zeros((24, H), jnp.float32).at[:hn, wd:wd + hd].set(hour_table)
    minute_p = jnp.zeros((8, H), jnp.float32).at[:mn, wd + hd:].set(minute_table)

    tw = t[:, :, 0, 1].reshape(-1).astype(jnp.int32)
    th = t[:, :, 0, 2].reshape(-1).astype(jnp.int32)
    tm = t[:, :, 0, 3].reshape(-1).astype(jnp.int32)

    grid_spec = pltpu.PrefetchScalarGridSpec(
        num_scalar_prefetch=3,
        grid=(steps,),
        in_specs=[
            pl.BlockSpec((N, H), lambda i, *_: (0, 0)),
            pl.BlockSpec((8, H), lambda i, *_: (0, 0)),
            pl.BlockSpec((24, H), lambda i, *_: (0, 0)),
            pl.BlockSpec((8, H), lambda i, *_: (0, 0)),
        ],
        out_specs=[
            pl.BlockSpec(memory_space=pl.ANY),
            pl.BlockSpec(memory_space=pl.ANY),
        ],
        scratch_shapes=[
            pltpu.VMEM((_G * N, H), jnp.float32),
            pltpu.VMEM((2, _G * N, H), jnp.float32),
            pltpu.SemaphoreType.DMA((_Q, 2)),
            pltpu.SemaphoreType.DMA((_Q, 2)),
        ],
    )
    xs, xt = pl.pallas_call(
        _body,
        grid_spec=grid_spec,
        out_shape=[jax.ShapeDtypeStruct((steps, _G * N, H), jnp.float32)] * 2,
        interpret=interpret,
    )(tw, th, tm, node_embedding, week_p, hour_p, minute_p)
    return xs.reshape(B, L, N, H), xt.reshape(B, L, N, H)
